# baseline (speedup 1.0000x reference)
import functools

import jax
import jax.numpy as jnp
from jax import lax
from jax.experimental import pallas as pl
from jax.experimental.pallas import tpu as pltpu
from jax.experimental.pallas import tpu_sc as plsc

NC = 2
NS = 16
NW = NC * NS

G = 100
K = 4
C = G * K
NBUF = 4
OBUF = 2


def _tok_pos_kernel(BF, L, E, per_w, n_chunks):
    mesh = plsc.VectorSubcoreMesh(core_axis_name="c", subcore_axis_name="s")
    idx_rows = per_w // G

    scratch = (
        [pltpu.VMEM((idx_rows, G), jnp.int32)]
        + [pltpu.VMEM((C, 2, 16), jnp.float32) for _ in range(NBUF)]
        + [pltpu.VMEM((C // 4, 128), jnp.float32) for _ in range(OBUF)]
        + [pltpu.VMEM((L, E), jnp.float32)]
        + [pltpu.SemaphoreType.DMA for _ in range(NBUF + OBUF)]
    )

    @functools.partial(
        pl.kernel,
        mesh=mesh,
        out_type=jax.ShapeDtypeStruct((BF // 4, 128), jnp.float32),
        compiler_params=pltpu.CompilerParams(use_tc_tiling_on_sc=False),
        scratch_types=scratch,
    )
    def k(x_hbm, tok_hbm, pos_hbm, out_hbm, idx_v, *rest):
        gbuf = rest[:NBUF]
        obuf = rest[NBUF:NBUF + OBUF]
        pos_v = rest[NBUF + OBUF]
        gsem = rest[NBUF + OBUF + 1:NBUF + OBUF + 1 + NBUF]
        ssem = rest[NBUF + OBUF + 1 + NBUF:]

        wid = lax.axis_index("s") * NC + lax.axis_index("c")
        base = wid * per_w
        pltpu.sync_copy(pos_hbm, pos_v)
        pltpu.sync_copy(
            x_hbm.at[pl.ds(pl.multiple_of(wid * idx_rows, 8), idx_rows)], idx_v
        )

        def gather_descr(c, b):
            return [
                pltpu.make_async_copy(
                    tok_hbm.at[idx_v.at[c * K + j]],
                    gbuf[b].at[pl.ds(j * G, G)],
                    gsem[b],
                )
                for j in range(K)
            ]

        def store_descr(c, ob):
            row0 = pl.multiple_of((base + c * C) // 4, C // 4)
            return pltpu.make_async_copy(
                obuf[ob], out_hbm.at[pl.ds(row0, C // 4)], ssem[ob]
            )

        for b in range(NBUF - 1):
            for d in gather_descr(b, b):
                d.start()

        def outer(t, carry):
            for phase in range(NBUF):
                c = t * NBUF + phase
                b = phase
                ob = phase % OBUF
                bn = (phase + NBUF - 1) % NBUF

                @pl.when(c + NBUF - 1 < n_chunks)
                def _fire():
                    for d in gather_descr(c + NBUF - 1, bn):
                        d.start()

                for d in gather_descr(c, b):
                    d.wait()

                @pl.when(c >= OBUF)
                def _drain():
                    store_descr(c - OBUF, ob).wait()

                gv = gbuf[b]
                ov = obuf[ob]

                def add_body(i, acc):
                    for u in range(4):
                        p = i * 4 + u
                        for h in range(2):
                            pv = pos_v[p, pl.ds(h * 16, 16)]
                            for s in range(C // L):
                                r = s * L + p
                                ov[s * (L // 4) + i, pl.ds(u * 32 + h * 16, 16)] = (
                                    gv[r, h, pl.ds(0, 16)] + pv
                                )
                    return acc

                lax.fori_loop(0, L // 4, add_body, 0)
                store_descr(c, ob).start()
            return carry

        lax.fori_loop(0, n_chunks // NBUF, outer, 0)

        for j in range(OBUF):
            c = n_chunks - OBUF + j
            store_descr(c, c % OBUF).wait()

    return k


def kernel(x, token_table, pos_table):
    B, L = x.shape
    V, E = token_table.shape
    BF = B * L
    per_w = BF // NW
    n_chunks = per_w // C

    xf = x.reshape(BF).astype(jnp.int32)
    x2 = xf.reshape(BF // G, G)
    tok3 = token_table.reshape(V, 2, 16)

    k = _tok_pos_kernel(BF, L, E, per_w, n_chunks)
    out = k(x2, tok3, pos_table)
    return out.reshape(B, L, E)


# speedup vs baseline: 2.6014x; 2.6014x over previous
import functools

import jax
import jax.numpy as jnp
from jax import lax
from jax.experimental import pallas as pl
from jax.experimental.pallas import tpu as pltpu
from jax.experimental.pallas import tpu_sc as plsc

NC = 2
NS = 16
NW = NC * NS

BB = 128
NBUF = 4
OB = 2


def _tok_pos_kernel(B, L, E, V):
    mesh = plsc.VectorSubcoreMesh(core_axis_name="c", subcore_axis_name="s")

    scratch = (
        [pltpu.VMEM((L, BB), jnp.int32)]
        + [pltpu.VMEM((BB, E), jnp.float32) for _ in range(NBUF)]
        + [pltpu.VMEM((E, BB), jnp.float32) for _ in range(OB)]
        + [pltpu.VMEM((L, E), jnp.float32)]
        + [pltpu.SemaphoreType.DMA for _ in range(NBUF + OB)]
    )

    @functools.partial(
        pl.kernel,
        mesh=mesh,
        out_type=jax.ShapeDtypeStruct((L, E // 8, B // BB, 8, BB), jnp.float32),
        compiler_params=pltpu.CompilerParams(
            use_tc_tiling_on_sc=False, needs_layout_passes=False
        ),
        scratch_types=scratch,
    )
    def k(xt_hbm, tok_hbm, pos_hbm, out_hbm, idx_v, *rest):
        gbuf = rest[:NBUF]
        obuf = rest[NBUF:NBUF + OB]
        pos_v = rest[NBUF + OB]
        gsem = rest[NBUF + OB + 1:NBUF + OB + 1 + NBUF]
        ssem = rest[NBUF + OB + 1 + NBUF:]

        wid = lax.axis_index("s") * NC + lax.axis_index("c")
        pltpu.sync_copy(pos_hbm, pos_v)
        pltpu.sync_copy(
            xt_hbm.at[:, pl.ds(pl.multiple_of(wid * BB, 8), BB)], idx_v
        )

        def gather_descr(p, b):
            return pltpu.make_async_copy(
                tok_hbm.at[idx_v.at[p]], gbuf[b], gsem[b]
            )

        def store_descr(p, ob):
            return [
                pltpu.make_async_copy(
                    obuf[ob].at[pl.ds(er * 8, 8)],
                    out_hbm.at[p, er, wid],
                    ssem[ob],
                )
                for er in range(E // 8)
            ]

        iota = lax.iota(jnp.int32, 16)

        for b in range(NBUF - 1):
            gather_descr(b, b).start()

        def outer(t, carry):
            for phase in range(NBUF):
                p = t * NBUF + phase
                b = phase
                ob = phase % OB
                bn = (phase + NBUF - 1) % NBUF

                @pl.when(p + NBUF - 1 < L)
                def _fire():
                    gather_descr(p + NBUF - 1, bn).start()

                gather_descr(p, b).wait()

                @pl.when(p >= OB)
                def _drain():
                    for d in store_descr(p - OB, ob):
                        d.wait()

                gv = gbuf[b]
                ov = obuf[ob]
                pv_splat = jnp.broadcast_to(p, (16,))

                def col_body(er, acc):
                    for ei in range(8):
                        e = er * 8 + ei
                        ev = jnp.broadcast_to(e, (16,))
                        pv = plsc.load_gather(pos_v, [pv_splat, ev])
                        for bb in range(BB // 16):
                            rowv = iota + bb * 16
                            val = plsc.load_gather(gv, [rowv, ev])
                            ov[e, pl.ds(bb * 16, 16)] = val + pv
                    return acc

                lax.fori_loop(0, E // 8, col_body, 0)
                for d in store_descr(p, ob):
                    d.start()
            return carry

        lax.fori_loop(0, L // NBUF, outer, 0)

        for j in range(OB):
            p = L - OB + j
            for d in store_descr(p, p % OB):
                d.wait()

    return k


def kernel(x, token_table, pos_table):
    B, L = x.shape
    V, E = token_table.shape

    xt = x.T.astype(jnp.int32)

    k = _tok_pos_kernel(B, L, E, V)
    o5 = k(xt, token_table, pos_table)
    return o5.transpose(2, 4, 0, 1, 3).reshape(B, L, E)


# speedup vs baseline: 3.2170x; 1.2367x over previous
import functools

import jax
import jax.numpy as jnp
from jax import lax
from jax.experimental import pallas as pl
from jax.experimental.pallas import tpu as pltpu
from jax.experimental.pallas import tpu_sc as plsc

NC = 2
NS = 16
NW = NC * NS

BB = 128
PITCH = 33
NBUF = 4
OB = 2


def _tok_pos_kernel(B, L, E, V):
    mesh = plsc.VectorSubcoreMesh(core_axis_name="c", subcore_axis_name="s")

    scratch = (
        [pltpu.VMEM((L // 8, 8, BB), jnp.int32)]
        + [pltpu.VMEM((BB, E), jnp.float32) for _ in range(NBUF)]
        + [pltpu.VMEM((BB * PITCH,), jnp.float32)]
        + [pltpu.VMEM((E, BB), jnp.float32) for _ in range(OB)]
        + [pltpu.VMEM((L, E), jnp.float32)]
        + [pltpu.SemaphoreType.DMA for _ in range(NBUF + OB)]
    )

    @functools.partial(
        pl.kernel,
        mesh=mesh,
        out_type=jax.ShapeDtypeStruct((L, E // 8, B // BB, 8, BB), jnp.float32),
        compiler_params=pltpu.CompilerParams(
            use_tc_tiling_on_sc=False, needs_layout_passes=False
        ),
        scratch_types=scratch,
    )
    def k(xn_hbm, tok_hbm, pos_hbm, out_hbm, idx_v, *rest):
        gbuf = rest[:NBUF]
        sbuf = rest[NBUF]
        obuf = rest[NBUF + 1:NBUF + 1 + OB]
        pos_v = rest[NBUF + 1 + OB]
        gsem = rest[NBUF + 2 + OB:NBUF + 2 + OB + NBUF]
        ssem = rest[NBUF + 2 + OB + NBUF:]

        wid = lax.axis_index("s") * NC + lax.axis_index("c")
        pltpu.sync_copy(pos_hbm, pos_v)
        pltpu.sync_copy(xn_hbm.at[:, wid], idx_v)

        def gather_descr(p, b):
            return pltpu.make_async_copy(
                tok_hbm.at[idx_v.at[p // 8, p % 8]], gbuf[b], gsem[b]
            )

        def store_descr(p, ob):
            return [
                pltpu.make_async_copy(
                    obuf[ob].at[pl.ds(er * 8, 8)],
                    out_hbm.at[p, er, wid],
                    ssem[ob],
                )
                for er in range(E // 8)
            ]

        iota = lax.iota(jnp.int32, 16)

        for b in range(NBUF - 1):
            gather_descr(b, b).start()

        def outer(t, carry):
            for phase in range(NBUF):
                p = t * NBUF + phase
                b = phase
                ob = phase % OB
                bn = (phase + NBUF - 1) % NBUF

                @pl.when(p + NBUF - 1 < L)
                def _fire():
                    gather_descr(p + NBUF - 1, bn).start()

                gather_descr(p, b).wait()

                @pl.when(p >= OB)
                def _drain():
                    for d in store_descr(p - OB, ob):
                        d.wait()

                gv = gbuf[b]
                ov = obuf[ob]
                pos0 = pos_v[p, pl.ds(0, 16)]
                pos1 = pos_v[p, pl.ds(16, 16)]

                def row_body(i, acc):
                    for u in range(8):
                        bi = i * 8 + u
                        sbuf[pl.ds(bi * PITCH, 16)] = gv[bi, pl.ds(0, 16)] + pos0
                        sbuf[pl.ds(bi * PITCH + 16, 16)] = (
                            gv[bi, pl.ds(16, 16)] + pos1
                        )
                    return acc

                lax.fori_loop(0, BB // 8, row_body, 0)

                def col_body(er, acc):
                    for ei in range(8):
                        e = er * 8 + ei
                        ev = jnp.broadcast_to(e, (16,))
                        for bb in range(BB // 16):
                            rowv = (iota + bb * 16) * PITCH + ev
                            val = plsc.load_gather(sbuf, [rowv])
                            ov[e, pl.ds(bb * 16, 16)] = val
                    return acc

                lax.fori_loop(0, E // 8, col_body, 0)
                for d in store_descr(p, ob):
                    d.start()
            return carry

        lax.fori_loop(0, L // NBUF, outer, 0)

        for j in range(OB):
            p = L - OB + j
            for d in store_descr(p, p % OB):
                d.wait()

    return k


def kernel(x, token_table, pos_table):
    B, L = x.shape
    V, E = token_table.shape

    xn = (
        x.T.astype(jnp.int32)
        .reshape(L // 8, 8, B // BB, BB)
        .transpose(0, 2, 1, 3)
    )

    k = _tok_pos_kernel(B, L, E, V)
    o5 = k(xn, token_table, pos_table)
    return o5.transpose(2, 4, 0, 1, 3).reshape(B, L, E)


# speedup vs baseline: 5.3132x; 1.6516x over previous
import functools

import jax
import jax.numpy as jnp
from jax import lax
from jax.experimental import pallas as pl
from jax.experimental.pallas import tpu as pltpu
from jax.experimental.pallas import tpu_sc as plsc

NC = 2
NS = 16
NW = NC * NS

BB = 128
PITCH = 33
NBUF = 4
OB = 2


def _tok_pos_kernel(B, L, E, V):
    mesh = plsc.VectorSubcoreMesh(core_axis_name="c", subcore_axis_name="s")

    scratch = (
        [pltpu.VMEM((L // 8, 8, BB), jnp.int32)]
        + [pltpu.VMEM((BB, E), jnp.float32) for _ in range(NBUF)]
        + [pltpu.VMEM((BB * PITCH,), jnp.float32)]
        + [pltpu.VMEM((E, BB), jnp.float32) for _ in range(OB)]
        + [pltpu.VMEM((L, E), jnp.float32)]
        + [pltpu.SemaphoreType.DMA for _ in range(NBUF + OB)]
    )

    @functools.partial(
        pl.kernel,
        mesh=mesh,
        out_type=jax.ShapeDtypeStruct((L, E // 8, B // BB, 8, BB), jnp.float32),
        compiler_params=pltpu.CompilerParams(
            use_tc_tiling_on_sc=False, needs_layout_passes=False
        ),
        scratch_types=scratch,
    )
    def k(xn_hbm, tok_hbm, pos_hbm, out_hbm, idx_v, *rest):
        gbuf = rest[:NBUF]
        sbuf = rest[NBUF]
        obuf = rest[NBUF + 1:NBUF + 1 + OB]
        pos_v = rest[NBUF + 1 + OB]
        gsem = rest[NBUF + 2 + OB:NBUF + 2 + OB + NBUF]
        ssem = rest[NBUF + 2 + OB + NBUF:]

        wid = lax.axis_index("s") * NC + lax.axis_index("c")
        pltpu.sync_copy(pos_hbm, pos_v)
        pltpu.sync_copy(xn_hbm.at[:, wid], idx_v)

        def gather_descr(p, b):
            return pltpu.make_async_copy(
                tok_hbm.at[idx_v.at[p // 8, p % 8]], gbuf[b], gsem[b]
            )

        def store_descr(p, ob):
            return [
                pltpu.make_async_copy(
                    obuf[ob].at[pl.ds(er * 8, 8)],
                    out_hbm.at[p, er, wid],
                    ssem[ob],
                )
                for er in range(E // 8)
            ]

        iota = lax.iota(jnp.int32, 16)

        for b in range(NBUF - 1):
            gather_descr(b, b).start()

        def outer(t, carry):
            for phase in range(NBUF):
                p = t * NBUF + phase
                b = phase
                ob = phase % OB
                bn = (phase + NBUF - 1) % NBUF

                @pl.when(p + NBUF - 1 < L)
                def _fire():
                    gather_descr(p + NBUF - 1, bn).start()

                gather_descr(p, b).wait()

                @pl.when(p >= OB)
                def _drain():
                    for d in store_descr(p - OB, ob):
                        d.wait()

                gv = gbuf[b]
                ov = obuf[ob]
                pos0 = pos_v[p, pl.ds(0, 16)]
                pos1 = pos_v[p, pl.ds(16, 16)]

                @plsc.parallel_loop(0, BB, 1, unroll=8)
                def row_body(bi):
                    sbuf[pl.ds(bi * PITCH, 16)] = gv[bi, pl.ds(0, 16)] + pos0
                    sbuf[pl.ds(bi * PITCH + 16, 16)] = gv[bi, pl.ds(16, 16)] + pos1

                rows33 = [(iota + bb * 16) * PITCH for bb in range(BB // 16)]

                @plsc.parallel_loop(0, E, 1, unroll=4)
                def col_body(e):
                    ev = jnp.broadcast_to(e, (16,))
                    for bb in range(BB // 16):
                        val = plsc.load_gather(sbuf, [rows33[bb] + ev])
                        ov[e, pl.ds(bb * 16, 16)] = val
                for d in store_descr(p, ob):
                    d.start()
            return carry

        lax.fori_loop(0, L // NBUF, outer, 0)

        for j in range(OB):
            p = L - OB + j
            for d in store_descr(p, p % OB):
                d.wait()

    return k


def kernel(x, token_table, pos_table):
    B, L = x.shape
    V, E = token_table.shape

    xn = (
        x.T.astype(jnp.int32)
        .reshape(L // 8, 8, B // BB, BB)
        .transpose(0, 2, 1, 3)
    )

    k = _tok_pos_kernel(B, L, E, V)
    o5 = k(xn, token_table, pos_table)
    return o5.transpose(2, 4, 0, 1, 3).reshape(B, L, E)


# speedup vs baseline: 14.9035x; 2.8050x over previous
import functools

import jax
import jax.numpy as jnp
from jax import lax
from jax.experimental import pallas as pl
from jax.experimental.pallas import tpu as pltpu
from jax.experimental.pallas import tpu_sc as plsc

NC = 2
NS = 16
NW = NC * NS

BB = 128
PITCH = 33
NBUF = 4
OB = 2


def _tok_pos_kernel(B, L, E, V):
    mesh = plsc.VectorSubcoreMesh(core_axis_name="c", subcore_axis_name="s")

    scratch = (
        [pltpu.VMEM((L // 8, 8, BB), jnp.int32)]
        + [pltpu.VMEM((BB, E), jnp.float32) for _ in range(NBUF)]
        + [pltpu.VMEM((BB * PITCH,), jnp.float32)]
        + [pltpu.VMEM((E, BB), jnp.float32) for _ in range(OB)]
        + [pltpu.VMEM((L, E), jnp.float32)]
        + [pltpu.SemaphoreType.DMA for _ in range(NBUF + OB)]
    )

    @functools.partial(
        pl.kernel,
        mesh=mesh,
        out_type=jax.ShapeDtypeStruct((L, E // 8, B // BB, 8, BB), jnp.float32),
        compiler_params=pltpu.CompilerParams(
            use_tc_tiling_on_sc=False, needs_layout_passes=False
        ),
        scratch_types=scratch,
    )
    def k(xn_hbm, tok_hbm, pos_hbm, out_hbm, idx_v, *rest):
        gbuf = rest[:NBUF]
        sbuf = rest[NBUF]
        obuf = rest[NBUF + 1:NBUF + 1 + OB]
        pos_v = rest[NBUF + 1 + OB]
        gsem = rest[NBUF + 2 + OB:NBUF + 2 + OB + NBUF]
        ssem = rest[NBUF + 2 + OB + NBUF:]

        wid = lax.axis_index("s") * NC + lax.axis_index("c")
        pltpu.sync_copy(pos_hbm, pos_v)
        pltpu.sync_copy(xn_hbm.at[:, wid], idx_v)

        def gather_descr(p, b):
            return pltpu.make_async_copy(
                tok_hbm.at[idx_v.at[p // 8, p % 8]], gbuf[b], gsem[b]
            )

        def store_descr(p, ob):
            return [
                pltpu.make_async_copy(
                    obuf[ob].at[pl.ds(er * 8, 8)],
                    out_hbm.at[p, er, wid],
                    ssem[ob],
                )
                for er in range(E // 8)
            ]

        iota = lax.iota(jnp.int32, 16)

        for b in range(NBUF - 1):
            gather_descr(b, b).start()

        def outer(t, carry):
            for phase in range(NBUF):
                p = t * NBUF + phase
                b = phase
                ob = phase % OB
                bn = (phase + NBUF - 1) % NBUF

                @pl.when(p + NBUF - 1 < L)
                def _fire():
                    gather_descr(p + NBUF - 1, bn).start()

                gather_descr(p, b).wait()

                @pl.when(p >= OB)
                def _drain():
                    for d in store_descr(p - OB, ob):
                        d.wait()

                gv = gbuf[b]
                ov = obuf[ob]
                pos0 = pos_v[p, pl.ds(0, 16)]
                pos1 = pos_v[p, pl.ds(16, 16)]

                @plsc.parallel_loop(0, BB, 1, unroll=8)
                def row_body(bi):
                    sbuf[pl.ds(bi * PITCH, 16)] = gv[bi, pl.ds(0, 16)] + pos0
                    sbuf[pl.ds(bi * PITCH + 16, 16)] = gv[bi, pl.ds(16, 16)] + pos1

                rows33 = [(iota + bb * 16) * PITCH for bb in range(BB // 16)]

                @plsc.parallel_loop(0, E, 1, unroll=4)
                def col_body(e):
                    ev = jnp.broadcast_to(e, (16,))
                    for bb in range(BB // 16):
                        val = plsc.load_gather(sbuf, [rows33[bb] + ev])
                        ov[e, pl.ds(bb * 16, 16)] = val
                for d in store_descr(p, ob):
                    d.start()
            return carry

        lax.fori_loop(0, L // NBUF, outer, 0)

        for j in range(OB):
            p = L - OB + j
            for d in store_descr(p, p % OB):
                d.wait()

    return k


TB = 128
DPITCH = 129
DNB = 4
DOB = 2


def _detile_kernel(V, E):
    mesh = plsc.VectorSubcoreMesh(core_axis_name="c", subcore_axis_name="s")
    n_full = V // TB
    base_cnt = n_full // NW
    extra = n_full - base_cnt * NW
    slots = base_cnt + 1
    slots += (-slots) % DNB
    tail = V - n_full * TB

    scratch = (
        [pltpu.VMEM((E, TB), jnp.float32) for _ in range(DNB)]
        + [pltpu.VMEM((E * DPITCH,), jnp.float32)]
        + [pltpu.VMEM((TB * E // 128, 128), jnp.float32) for _ in range(DOB)]
        + [pltpu.SemaphoreType.DMA for _ in range(DNB + DOB)]
    )

    @functools.partial(
        pl.kernel,
        mesh=mesh,
        out_type=jax.ShapeDtypeStruct((V * E // 128, 128), jnp.float32),
        compiler_params=pltpu.CompilerParams(needs_layout_passes=False),
        scratch_types=scratch,
    )
    def k(tt_hbm, tail_hbm, out_hbm, *rest):
        tbuf = rest[:DNB]
        sbuf = rest[DNB]
        obuf = rest[DNB + 1:DNB + 1 + DOB]
        lsem = rest[DNB + 1 + DOB:DNB + 1 + DOB + DNB]
        osem = rest[DNB + 1 + DOB + DNB:]

        wid = lax.axis_index("s") * NC + lax.axis_index("c")
        cnt = base_cnt + (wid < extra).astype(jnp.int32)
        start = wid * base_cnt + jnp.minimum(wid, extra)

        iota = lax.iota(jnp.int32, 16)

        def load_descr(i, b):
            c = pl.multiple_of((start + i) * TB, TB)
            return pltpu.make_async_copy(
                tt_hbm.at[:, pl.ds(c, TB)], tbuf[b], lsem[b]
            )

        def store_descr(i, ob):
            r = pl.multiple_of((start + i) * (TB * E // 128), TB * E // 128)
            return pltpu.make_async_copy(
                obuf[ob], out_hbm.at[pl.ds(r, TB * E // 128)], osem[ob]
            )

        for b in range(DNB - 1):
            @pl.when(b < cnt)
            def _prime():
                load_descr(b, b).start()

        def outer(t, carry):
            for phase in range(DNB):
                i = t * DNB + phase
                b = phase
                ob = phase % DOB
                bn = (phase + DNB - 1) % DNB

                @pl.when(i + DNB - 1 < cnt)
                def _fire():
                    load_descr(i + DNB - 1, bn).start()

                @pl.when(i < cnt)
                def _work():
                    load_descr(i, b).wait()

                    @pl.when(i >= DOB)
                    def _drain():
                        store_descr(i - DOB, ob).wait()

                    tv = tbuf[b]
                    ov = obuf[ob]

                    @plsc.parallel_loop(0, E, 1, unroll=4)
                    def stage(e):
                        for g in range(TB // 16):
                            sbuf[pl.ds(e * DPITCH + g * 16, 16)] = tv[
                                e, pl.ds(g * 16, 16)
                            ]

                    rows = [(h * 16 + iota) * DPITCH for h in range(E // 16)]

                    @plsc.parallel_loop(0, TB, 1, unroll=4)
                    def emit(v):
                        vv = jnp.broadcast_to(v, (16,))
                        for h in range(E // 16):
                            val = plsc.load_gather(sbuf, [rows[h] + vv])
                            ov[
                                v // (128 // E),
                                pl.ds((v % (128 // E)) * E + h * 16, 16),
                            ] = val

                    store_descr(i, ob).start()
            return carry

        lax.fori_loop(0, slots // DNB, outer, 0)

        for j in range(DOB):
            @pl.when(cnt - DOB + j >= 0)
            def _final():
                i = cnt - DOB + j
                for ob in range(DOB):
                    @pl.when((i % DOB) == ob)
                    def _w():
                        store_descr(i, ob).wait()

        @pl.when(wid == NW - 1)
        def _tail():
            ov = obuf[0]
            pltpu.sync_copy(tail_hbm, ov.at[pl.ds(0, tail * E // 128)])
            pltpu.sync_copy(
                ov.at[pl.ds(0, tail * E // 128)],
                out_hbm.at[
                    pl.ds(
                        pl.multiple_of(n_full * TB * E // 128, 8),
                        tail * E // 128,
                    )
                ],
            )

    return k


def kernel(x, token_table, pos_table):
    B, L = x.shape
    V, E = token_table.shape

    xn = (
        x.T.astype(jnp.int32)
        .reshape(L // 8, 8, B // BB, BB)
        .transpose(0, 2, 1, 3)
    )

    dk = _detile_kernel(V, E)
    n_full_tok = (V // TB) * TB
    tok_tail = token_table[n_full_tok:].reshape((V - n_full_tok) * E // 128, 128)
    tok_lin = dk(token_table.T, tok_tail).reshape(V, E)

    k = _tok_pos_kernel(B, L, E, V)
    o5 = k(xn, tok_lin, pos_table)
    return o5.transpose(2, 4, 0, 1, 3).reshape(B, L, E)


# speedup vs baseline: 15.1073x; 1.0137x over previous
import functools

import jax
import jax.numpy as jnp
from jax import lax
from jax.experimental import pallas as pl
from jax.experimental.pallas import tpu as pltpu
from jax.experimental.pallas import tpu_sc as plsc

NC = 2
NS = 16
NW = NC * NS

BB = 128
PITCH = 33
NBUF = 8
OB = 2


def _tok_pos_kernel(B, L, E, V):
    mesh = plsc.VectorSubcoreMesh(core_axis_name="c", subcore_axis_name="s")

    scratch = (
        [pltpu.VMEM((L // 8, 8, BB), jnp.int32)]
        + [pltpu.VMEM((BB, E), jnp.float32) for _ in range(NBUF)]
        + [pltpu.VMEM((BB * PITCH,), jnp.float32)]
        + [pltpu.VMEM((E, BB), jnp.float32) for _ in range(OB)]
        + [pltpu.VMEM((L, E), jnp.float32)]
        + [pltpu.SemaphoreType.DMA for _ in range(NBUF + OB)]
    )

    @functools.partial(
        pl.kernel,
        mesh=mesh,
        out_type=jax.ShapeDtypeStruct((L, E // 8, B // BB, 8, BB), jnp.float32),
        compiler_params=pltpu.CompilerParams(
            use_tc_tiling_on_sc=False, needs_layout_passes=False
        ),
        scratch_types=scratch,
    )
    def k(xn_hbm, tok_hbm, pos_hbm, out_hbm, idx_v, *rest):
        gbuf = rest[:NBUF]
        sbuf = rest[NBUF]
        obuf = rest[NBUF + 1:NBUF + 1 + OB]
        pos_v = rest[NBUF + 1 + OB]
        gsem = rest[NBUF + 2 + OB:NBUF + 2 + OB + NBUF]
        ssem = rest[NBUF + 2 + OB + NBUF:]

        wid = lax.axis_index("s") * NC + lax.axis_index("c")
        pltpu.sync_copy(pos_hbm, pos_v)
        pltpu.sync_copy(xn_hbm.at[:, wid], idx_v)

        def gather_descr(p, b):
            return pltpu.make_async_copy(
                tok_hbm.at[idx_v.at[p // 8, p % 8]], gbuf[b], gsem[b]
            )

        def store_descr(p, ob):
            return [
                pltpu.make_async_copy(
                    obuf[ob].at[pl.ds(er * 8, 8)],
                    out_hbm.at[p, er, wid],
                    ssem[ob],
                )
                for er in range(E // 8)
            ]

        iota = lax.iota(jnp.int32, 16)

        for b in range(NBUF - 1):
            gather_descr(b, b).start()

        def outer(t, carry):
            for phase in range(NBUF):
                p = t * NBUF + phase
                b = phase
                ob = phase % OB
                bn = (phase + NBUF - 1) % NBUF

                @pl.when(p + NBUF - 1 < L)
                def _fire():
                    gather_descr(p + NBUF - 1, bn).start()

                gather_descr(p, b).wait()

                @pl.when(p >= OB)
                def _drain():
                    for d in store_descr(p - OB, ob):
                        d.wait()

                gv = gbuf[b]
                ov = obuf[ob]
                pos0 = pos_v[p, pl.ds(0, 16)]
                pos1 = pos_v[p, pl.ds(16, 16)]

                @plsc.parallel_loop(0, BB, 1, unroll=8)
                def row_body(bi):
                    sbuf[pl.ds(bi * PITCH, 16)] = gv[bi, pl.ds(0, 16)] + pos0
                    sbuf[pl.ds(bi * PITCH + 16, 16)] = gv[bi, pl.ds(16, 16)] + pos1

                rows33 = [(iota + bb * 16) * PITCH for bb in range(BB // 16)]

                @plsc.parallel_loop(0, E, 1, unroll=4)
                def col_body(e):
                    ev = jnp.broadcast_to(e, (16,))
                    for bb in range(BB // 16):
                        val = plsc.load_gather(sbuf, [rows33[bb] + ev])
                        ov[e, pl.ds(bb * 16, 16)] = val
                for d in store_descr(p, ob):
                    d.start()
            return carry

        lax.fori_loop(0, L // NBUF, outer, 0)

        for j in range(OB):
            p = L - OB + j
            for d in store_descr(p, p % OB):
                d.wait()

    return k


TB = 128
DPITCH = 129
DNB = 6
DOB = 2


def _detile_kernel(V, E):
    mesh = plsc.VectorSubcoreMesh(core_axis_name="c", subcore_axis_name="s")
    n_full = V // TB
    base_cnt = n_full // NW
    extra = n_full - base_cnt * NW
    slots = base_cnt + 1
    slots += (-slots) % DNB
    tail = V - n_full * TB

    scratch = (
        [pltpu.VMEM((E, TB), jnp.float32) for _ in range(DNB)]
        + [pltpu.VMEM((E * DPITCH,), jnp.float32)]
        + [pltpu.VMEM((TB * E // 128, 128), jnp.float32) for _ in range(DOB)]
        + [pltpu.SemaphoreType.DMA for _ in range(DNB + DOB)]
    )

    @functools.partial(
        pl.kernel,
        mesh=mesh,
        out_type=jax.ShapeDtypeStruct((V * E // 128, 128), jnp.float32),
        compiler_params=pltpu.CompilerParams(needs_layout_passes=False),
        scratch_types=scratch,
    )
    def k(tt_hbm, tail_hbm, out_hbm, *rest):
        tbuf = rest[:DNB]
        sbuf = rest[DNB]
        obuf = rest[DNB + 1:DNB + 1 + DOB]
        lsem = rest[DNB + 1 + DOB:DNB + 1 + DOB + DNB]
        osem = rest[DNB + 1 + DOB + DNB:]

        wid = lax.axis_index("s") * NC + lax.axis_index("c")
        cnt = base_cnt + (wid < extra).astype(jnp.int32)
        start = wid * base_cnt + jnp.minimum(wid, extra)

        iota = lax.iota(jnp.int32, 16)

        def load_descr(i, b):
            c = pl.multiple_of((start + i) * TB, TB)
            return pltpu.make_async_copy(
                tt_hbm.at[:, pl.ds(c, TB)], tbuf[b], lsem[b]
            )

        def store_descr(i, ob):
            r = pl.multiple_of((start + i) * (TB * E // 128), TB * E // 128)
            return pltpu.make_async_copy(
                obuf[ob], out_hbm.at[pl.ds(r, TB * E // 128)], osem[ob]
            )

        for b in range(DNB - 1):
            @pl.when(b < cnt)
            def _prime():
                load_descr(b, b).start()

        def outer(t, carry):
            for phase in range(DNB):
                i = t * DNB + phase
                b = phase
                ob = phase % DOB
                bn = (phase + DNB - 1) % DNB

                @pl.when(i + DNB - 1 < cnt)
                def _fire():
                    load_descr(i + DNB - 1, bn).start()

                @pl.when(i < cnt)
                def _work():
                    load_descr(i, b).wait()

                    @pl.when(i >= DOB)
                    def _drain():
                        store_descr(i - DOB, ob).wait()

                    tv = tbuf[b]
                    ov = obuf[ob]

                    @plsc.parallel_loop(0, E, 1, unroll=4)
                    def stage(e):
                        for g in range(TB // 16):
                            sbuf[pl.ds(e * DPITCH + g * 16, 16)] = tv[
                                e, pl.ds(g * 16, 16)
                            ]

                    rows = [(h * 16 + iota) * DPITCH for h in range(E // 16)]

                    @plsc.parallel_loop(0, TB, 1, unroll=4)
                    def emit(v):
                        vv = jnp.broadcast_to(v, (16,))
                        for h in range(E // 16):
                            val = plsc.load_gather(sbuf, [rows[h] + vv])
                            ov[
                                v // (128 // E),
                                pl.ds((v % (128 // E)) * E + h * 16, 16),
                            ] = val

                    store_descr(i, ob).start()
            return carry

        lax.fori_loop(0, slots // DNB, outer, 0)

        for j in range(DOB):
            @pl.when(cnt - DOB + j >= 0)
            def _final():
                i = cnt - DOB + j
                for ob in range(DOB):
                    @pl.when((i % DOB) == ob)
                    def _w():
                        store_descr(i, ob).wait()

        @pl.when(wid == NW - 1)
        def _tail():
            ov = obuf[0]
            pltpu.sync_copy(tail_hbm, ov.at[pl.ds(0, tail * E // 128)])
            pltpu.sync_copy(
                ov.at[pl.ds(0, tail * E // 128)],
                out_hbm.at[
                    pl.ds(
                        pl.multiple_of(n_full * TB * E // 128, 8),
                        tail * E // 128,
                    )
                ],
            )

    return k


def kernel(x, token_table, pos_table):
    B, L = x.shape
    V, E = token_table.shape

    xn = (
        x.T.astype(jnp.int32)
        .reshape(L // 8, 8, B // BB, BB)
        .transpose(0, 2, 1, 3)
    )

    dk = _detile_kernel(V, E)
    n_full_tok = (V // TB) * TB
    tok_tail = token_table[n_full_tok:].reshape((V - n_full_tok) * E // 128, 128)
    tok_lin = dk(token_table.T, tok_tail).reshape(V, E)

    k = _tok_pos_kernel(B, L, E, V)
    o5 = k(xn, tok_lin, pos_table)
    return o5.transpose(2, 4, 0, 1, 3).reshape(B, L, E)
